# baseline (speedup 1.0000x reference)
import functools

import jax
import jax.numpy as jnp
from jax import lax
from jax.experimental import pallas as pl
from jax.experimental.pallas import tpu as pltpu
from jax.experimental.pallas import tpu_sc as plsc

_SHAPE = (64, 128)
_N = _SHAPE[0] * _SHAPE[1]
_NC, _NS, _L = 2, 16, 16
_NW = _NC * _NS
_SEG = _N // _NW
_CHUNKS = _SEG // _L
_NWORD = _N // 4

_BETA = 0.9


def _tc_body(spk_ref, x_ref, act_ref, gain_ref, thr_ref, lat_ref,
             out_ref, any_ref):
    gg = gain_ref[...]
    ig = gg + (1.0 - gg) * 0.2
    act = _BETA * act_ref[...] + (x_ref[...] + lat_ref[...]) * ig + 0.05
    out_ref[...] = jnp.where(act > thr_ref[...], 1.0, 0.0)
    any_ref[0, 0] = jnp.sum(spk_ref[...].astype(jnp.int32))


_tc_kernel = pl.pallas_call(
    _tc_body,
    out_shape=(
        jax.ShapeDtypeStruct(_SHAPE, jnp.float32),
        jax.ShapeDtypeStruct((1, 1), jnp.int32),
    ),
    out_specs=(
        pl.BlockSpec(memory_space=pltpu.VMEM),
        pl.BlockSpec(memory_space=pltpu.SMEM),
    ),
)


def _sc_body(sw_hbm, x_hbm, act_hbm, gain_hbm, thr_hbm, w_hbm, out_hbm,
             o_v, sem2):
    wid = lax.axis_index("s") * _NC + lax.axis_index("c")
    base = wid * _SEG
    co = pltpu.async_copy(x_hbm.at[pl.ds(base, _SEG)], o_v, sem2)
    co.wait()
    pltpu.sync_copy(o_v, out_hbm.at[pl.ds(base, _SEG)])


_sc_kernel = functools.partial(
    pl.kernel,
    out_type=jax.ShapeDtypeStruct((_N,), jnp.float32),
    mesh=plsc.VectorSubcoreMesh(core_axis_name="c", subcore_axis_name="s",
                                num_cores=_NC, num_subcores=_NS),
    scratch_types=[
        pltpu.VMEM((_SEG,), jnp.float32),
        pltpu.SemaphoreType.DMA,
    ],
)(_sc_body)


def kernel(x, activation, input_gain, threshold, freq_act, lateral_weights,
           spikes):
    del freq_act

    zeros_lat = jnp.zeros(_SHAPE, jnp.float32)
    out0, nspk = _tc_kernel(spikes, x, activation, input_gain, threshold,
                            zeros_lat)

    def spike_branch(_):
        sw = lax.bitcast_convert_type(
            spikes.reshape(_NWORD, 4).astype(jnp.int8), jnp.int32)
        out = _sc_kernel(sw, x.reshape(_N), activation.reshape(_N),
                         input_gain.reshape(_N), threshold.reshape(_N),
                         lateral_weights)
        return out.reshape(_SHAPE)

    def empty_branch(_):
        return out0

    outf = lax.cond(nspk[0, 0] > 0, spike_branch, empty_branch, 0)
    return outf.astype(jnp.bool_)


# speedup vs baseline: 1.0108x; 1.0108x over previous
import functools

import jax
import jax.numpy as jnp
from jax import lax
from jax.experimental import pallas as pl
from jax.experimental.pallas import tpu as pltpu
from jax.experimental.pallas import tpu_sc as plsc

_SHAPE = (64, 128)
_N = _SHAPE[0] * _SHAPE[1]
_NC, _NS, _L = 2, 16, 16
_NW = _NC * _NS
_SEG = _N // _NW
_CHUNKS = _SEG // _L
_NWORD = _N // 4

_BETA = 0.9


def _tc_body(spk_ref, x_ref, act_ref, gain_ref, thr_ref, out_ref, cnt_ref):
    gg = gain_ref[...]
    ig = gg + (1.0 - gg) * 0.2
    act = _BETA * act_ref[...] + x_ref[...] * ig + 0.05
    out_ref[...] = act > thr_ref[...]
    cnt_ref[0, 0] = jnp.sum(spk_ref[...].astype(jnp.int32))


_tc_kernel = pl.pallas_call(
    _tc_body,
    out_shape=(
        jax.ShapeDtypeStruct(_SHAPE, jnp.bool_),
        jax.ShapeDtypeStruct((1, 1), jnp.int32),
    ),
    out_specs=(
        pl.BlockSpec(memory_space=pltpu.VMEM),
        pl.BlockSpec(memory_space=pltpu.SMEM),
    ),
)


def _sc_body(sw_hbm, x_hbm, act_hbm, gain_hbm, thr_hbm, w_hbm, out_hbm,
             sp_v, row_v, acc_v, x_v, a_v, g_v, t_v, o_v, sem, sem2):
    wid = lax.axis_index("s") * _NC + lax.axis_index("c")
    base = wid * _SEG

    pltpu.sync_copy(sw_hbm, sp_v.at[pl.ds(0, _NWORD)])
    cx = pltpu.async_copy(x_hbm.at[pl.ds(base, _SEG)], x_v, sem2)
    ca = pltpu.async_copy(act_hbm.at[pl.ds(base, _SEG)], a_v, sem2)
    cg = pltpu.async_copy(gain_hbm.at[pl.ds(base, _SEG)], g_v, sem2)
    ct = pltpu.async_copy(thr_hbm.at[pl.ds(base, _SEG)], t_v, sem2)

    def zero_body(k, c):
        acc_v[pl.ds(k * _L, _L)] = jnp.zeros((_L,), jnp.float32)
        return c

    lax.fori_loop(0, _CHUNKS, zero_body, 0)

    def word_body(q, carry):
        w = sp_v[pl.ds(q, _L)][0]

        @pl.when(w != 0)
        def _():
            for bidx in range(4):
                @pl.when(((w >> (8 * bidx)) & 0xFF) != 0)
                def _():
                    pltpu.sync_copy(
                        w_hbm.at[q * 4 + bidx, pl.ds(base, _SEG)], row_v)

                    def add_chunk(k, cc):
                        sl = pl.ds(k * _L, _L)
                        acc_v[sl] = acc_v[sl] + row_v[sl]
                        return cc
                    lax.fori_loop(0, _CHUNKS, add_chunk, 0)
        return carry

    lax.fori_loop(0, _NWORD, word_body, 0)

    cx.wait()
    ca.wait()
    cg.wait()
    ct.wait()

    def ew_body(k, c):
        sl = pl.ds(k * _L, _L)
        gg = g_v[sl]
        ig = gg + (1.0 - gg) * 0.2
        act = _BETA * a_v[sl] + (x_v[sl] + acc_v[sl]) * ig + 0.05
        o_v[sl] = jnp.where(act > t_v[sl], 1.0, 0.0)
        return c

    lax.fori_loop(0, _CHUNKS, ew_body, 0)
    pltpu.sync_copy(o_v, out_hbm.at[pl.ds(base, _SEG)])


_sc_kernel = functools.partial(
    pl.kernel,
    out_type=jax.ShapeDtypeStruct((_N,), jnp.float32),
    mesh=plsc.VectorSubcoreMesh(core_axis_name="c", subcore_axis_name="s",
                                num_cores=_NC, num_subcores=_NS),
    scratch_types=[
        pltpu.VMEM((_NWORD + _L,), jnp.int32),
        pltpu.VMEM((_SEG,), jnp.float32),
        pltpu.VMEM((_SEG,), jnp.float32),
        pltpu.VMEM((_SEG,), jnp.float32),
        pltpu.VMEM((_SEG,), jnp.float32),
        pltpu.VMEM((_SEG,), jnp.float32),
        pltpu.VMEM((_SEG,), jnp.float32),
        pltpu.VMEM((_SEG,), jnp.float32),
        pltpu.SemaphoreType.DMA,
        pltpu.SemaphoreType.DMA,
    ],
)(_sc_body)


def kernel(x, activation, input_gain, threshold, freq_act, lateral_weights,
           spikes):
    del freq_act

    out0, nspk = _tc_kernel(spikes, x, activation, input_gain, threshold)

    def spike_branch(_):
        sw = lax.bitcast_convert_type(
            spikes.reshape(_NWORD, 4).astype(jnp.int8), jnp.int32)
        out = _sc_kernel(sw, x.reshape(_N), activation.reshape(_N),
                         input_gain.reshape(_N), threshold.reshape(_N),
                         lateral_weights)
        return out.reshape(_SHAPE) != 0.0

    def empty_branch(_):
        return out0

    return lax.cond(nspk[0, 0] > 0, spike_branch, empty_branch, 0)


# speedup vs baseline: 1.0773x; 1.0658x over previous
import functools

import jax
import jax.numpy as jnp
from jax import lax
from jax.experimental import pallas as pl
from jax.experimental.pallas import tpu as pltpu
from jax.experimental.pallas import tpu_sc as plsc

_SHAPE = (64, 128)
_N = _SHAPE[0] * _SHAPE[1]
_NC, _NS, _L = 2, 16, 16
_NW = _NC * _NS
_SEG = _N // _NW
_CHUNKS = _SEG // _L
_NWORD = _N // 4

_BETA = 0.9


def _tc_body(spk_ref, x_ref, act_ref, gain_ref, thr_ref, out_ref, cnt_ref,
             spk32_ref):
    gg = gain_ref[...]
    ig = gg + (1.0 - gg) * 0.2
    act = _BETA * act_ref[...] + x_ref[...] * ig + 0.05
    out_ref[...] = act > thr_ref[...]
    si = spk_ref[...].astype(jnp.int32)
    spk32_ref[...] = si
    cnt_ref[0, 0] = jnp.sum(si)


_tc_kernel = pl.pallas_call(
    _tc_body,
    out_shape=(
        jax.ShapeDtypeStruct(_SHAPE, jnp.bool_),
        jax.ShapeDtypeStruct((1, 1), jnp.int32),
        jax.ShapeDtypeStruct(_SHAPE, jnp.int32),
    ),
    out_specs=(
        pl.BlockSpec(memory_space=pltpu.VMEM),
        pl.BlockSpec(memory_space=pltpu.SMEM),
        pl.BlockSpec(memory_space=pltpu.VMEM),
    ),
)


def _sc_body(sw_hbm, x_hbm, act_hbm, gain_hbm, thr_hbm, w_hbm, out_hbm,
             sp_v, row_v, acc_v, x_v, a_v, g_v, t_v, o_v, sem, sem2):
    wid = lax.axis_index("s") * _NC + lax.axis_index("c")
    base = wid * _SEG

    pltpu.sync_copy(sw_hbm, sp_v.at[pl.ds(0, _N)])
    cx = pltpu.async_copy(x_hbm.at[pl.ds(base, _SEG)], x_v, sem2)
    ca = pltpu.async_copy(act_hbm.at[pl.ds(base, _SEG)], a_v, sem2)
    cg = pltpu.async_copy(gain_hbm.at[pl.ds(base, _SEG)], g_v, sem2)
    ct = pltpu.async_copy(thr_hbm.at[pl.ds(base, _SEG)], t_v, sem2)

    def zero_body(k, c):
        acc_v[pl.ds(k * _L, _L)] = jnp.zeros((_L,), jnp.float32)
        return c

    lax.fori_loop(0, _CHUNKS, zero_body, 0)

    def row_body(r, carry):
        s = sp_v[pl.ds(r, _L)][0]

        @pl.when(s != 0)
        def _():
            pltpu.sync_copy(w_hbm.at[r, pl.ds(base, _SEG)], row_v)

            def add_chunk(k, cc):
                sl = pl.ds(k * _L, _L)
                acc_v[sl] = acc_v[sl] + row_v[sl]
                return cc
            lax.fori_loop(0, _CHUNKS, add_chunk, 0)
        return carry

    lax.fori_loop(0, _N, row_body, 0)

    cx.wait()
    ca.wait()
    cg.wait()
    ct.wait()

    def ew_body(k, c):
        sl = pl.ds(k * _L, _L)
        gg = g_v[sl]
        ig = gg + (1.0 - gg) * 0.2
        act = _BETA * a_v[sl] + (x_v[sl] + acc_v[sl]) * ig + 0.05
        o_v[sl] = jnp.where(act > t_v[sl], 1.0, 0.0)
        return c

    lax.fori_loop(0, _CHUNKS, ew_body, 0)
    pltpu.sync_copy(o_v, out_hbm.at[pl.ds(base, _SEG)])


_sc_kernel = functools.partial(
    pl.kernel,
    out_type=jax.ShapeDtypeStruct((_N,), jnp.float32),
    mesh=plsc.VectorSubcoreMesh(core_axis_name="c", subcore_axis_name="s",
                                num_cores=_NC, num_subcores=_NS),
    scratch_types=[
        pltpu.VMEM((_N + _L,), jnp.int32),
        pltpu.VMEM((_SEG,), jnp.float32),
        pltpu.VMEM((_SEG,), jnp.float32),
        pltpu.VMEM((_SEG,), jnp.float32),
        pltpu.VMEM((_SEG,), jnp.float32),
        pltpu.VMEM((_SEG,), jnp.float32),
        pltpu.VMEM((_SEG,), jnp.float32),
        pltpu.VMEM((_SEG,), jnp.float32),
        pltpu.SemaphoreType.DMA,
        pltpu.SemaphoreType.DMA,
    ],
)(_sc_body)


def kernel(x, activation, input_gain, threshold, freq_act, lateral_weights,
           spikes):
    del freq_act

    out0, nspk, spk32 = _tc_kernel(spikes, x, activation, input_gain,
                                   threshold)

    def spike_branch(_):
        sw = spk32.reshape(_N)
        out = _sc_kernel(sw, x.reshape(_N), activation.reshape(_N),
                         input_gain.reshape(_N), threshold.reshape(_N),
                         lateral_weights)
        return out.reshape(_SHAPE) != 0.0

    def empty_branch(_):
        return out0

    return lax.cond(nspk[0, 0] > 0, spike_branch, empty_branch, 0)
